# baseline (speedup 1.0000x reference)
import jax
import jax.numpy as jnp
from jax import lax
from jax.experimental import pallas as pl
from jax.experimental.pallas import tpu as pltpu
from jax.experimental.pallas import tpu_sc as plsc

N = 1048576
D = 8
NC, NS = 2, 16
NW = NC * NS
BPW = N // NW
CHUNK = 2048
GB = 128
K = CHUNK // GB
NCHUNK = BPW // CHUNK


def _sc_body(rays_hbm, idx_hbm, p_hbm, v_hbm, idx_v, rows_v, p_v, v_v, gsem):
    c = lax.axis_index("c")
    s = lax.axis_index("s")
    wid = s * NC + c
    pltpu.sync_copy(idx_hbm.at[pl.ds(wid * (BPW // GB), BPW // GB)], idx_v)

    lane = lax.iota(jnp.int32, 16)

    def chunk_body(ci, carry):
        handles = []
        for j in range(K):
            handles.append(
                pltpu.async_copy(
                    rays_hbm.at[idx_v.at[ci * K + j]],
                    rows_v.at[pl.ds(j * GB, GB)],
                    gsem,
                )
            )
        for h in handles:
            h.wait()

        def group(g, carry2):
            rows_r = g * 16 + lane

            def col(cc):
                return plsc.load_gather(
                    rows_v, [rows_r, jnp.full((16,), cc, jnp.int32)]
                )

            r0 = col(0)
            r1 = col(1)
            r3 = col(3)
            r4 = col(4)
            r5 = col(5)

            ssq = r3 * r3 + r4 * r4 + r5 * r5
            seed = plsc.bitcast(
                jnp.int32(0x5F3759DF) - lax.shift_right_logical(
                    plsc.bitcast(ssq, jnp.int32), 1
                ),
                jnp.float32,
            )
            half = 0.5 * ssq
            y = seed * (1.5 - half * seed * seed)
            y = y * (1.5 - half * y * y)
            y = y * (1.5 - half * y * y)
            inv = y

            oflat = rows_r * 3

            def put(dst, cc, val):
                plsc.store_scatter(dst, [oflat + cc], val)

            put(p_v, 0, jnp.zeros((16,), jnp.float32))
            put(p_v, 1, 1000.0 * r0)
            put(p_v, 2, 1000.0 * r1)
            put(v_v, 0, -r5 * inv)
            put(v_v, 1, r3 * inv)
            put(v_v, 2, r4 * inv)
            return carry2

        lax.fori_loop(0, CHUNK // 16, group, 0)

        base = (wid * BPW + ci * CHUNK) * 3
        pltpu.sync_copy(p_v, p_hbm.at[pl.ds(base, CHUNK * 3)])
        pltpu.sync_copy(v_v, v_hbm.at[pl.ds(base, CHUNK * 3)])
        return carry

    lax.fori_loop(0, NCHUNK, chunk_body, 0)


_sc_call = pl.kernel(
    _sc_body,
    out_type=(
        jax.ShapeDtypeStruct((N * 3,), jnp.float32),
        jax.ShapeDtypeStruct((N * 3,), jnp.float32),
    ),
    mesh=plsc.VectorSubcoreMesh(core_axis_name="c", subcore_axis_name="s"),
    compiler_params=pltpu.CompilerParams(
        needs_layout_passes=False, use_tc_tiling_on_sc=False
    ),
    scratch_types=[
        pltpu.VMEM((BPW // GB, GB), jnp.int32),
        pltpu.VMEM((CHUNK, D), jnp.float32),
        pltpu.VMEM((CHUNK * 3,), jnp.float32),
        pltpu.VMEM((CHUNK * 3,), jnp.float32),
        pltpu.SemaphoreType.DMA,
    ],
)


def kernel(all_rays, indices):
    rays8 = jnp.concatenate(
        [all_rays, jnp.zeros((all_rays.shape[0], 2), jnp.float32)], axis=1
    )
    idx2 = indices.reshape(N // GB, GB)
    p_flat, v_flat = _sc_call(rays8, idx2)
    return (p_flat.reshape(N, 3), v_flat.reshape(N, 3))


# speedup vs baseline: 1.1953x; 1.1953x over previous
import jax
import jax.numpy as jnp
from jax import lax
from jax.experimental import pallas as pl
from jax.experimental.pallas import tpu as pltpu
from jax.experimental.pallas import tpu_sc as plsc

N = 1048576
D = 8
NC, NS = 2, 16
NW = NC * NS
BPW = N // NW
CHUNK = 2048
GB = 128
K = CHUNK // GB
NCHUNK = BPW // CHUNK


def _sc_body(rays_hbm, idx_hbm, p_hbm, v_hbm, idx_v, rows_v, p_v, v_v, gsem):
    c = lax.axis_index("c")
    s = lax.axis_index("s")
    wid = s * NC + c
    pltpu.sync_copy(idx_hbm.at[pl.ds(wid * (BPW // GB), BPW // GB)], idx_v)

    lane = lax.iota(jnp.int32, 16)

    def chunk_body(ci, carry):
        handles = []
        for j in range(K):
            handles.append(
                pltpu.async_copy(
                    rays_hbm.at[idx_v.at[ci * K + j]],
                    rows_v.at[pl.ds(j * GB, GB)],
                    gsem,
                )
            )
        for h in handles:
            h.wait()

        def group(g, carry2):
            rows_r = g * 16 + lane

            def col(cc):
                return plsc.load_gather(
                    rows_v, [rows_r, jnp.full((16,), cc, jnp.int32)]
                )

            r0 = col(0)
            r1 = col(1)
            r3 = col(3)
            r4 = col(4)
            r5 = col(5)

            ssq = r3 * r3 + r4 * r4 + r5 * r5
            seed = plsc.bitcast(
                jnp.int32(0x5F3759DF) - lax.shift_right_logical(
                    plsc.bitcast(ssq, jnp.int32), 1
                ),
                jnp.float32,
            )
            half = 0.5 * ssq
            y = seed * (1.5 - half * seed * seed)
            y = y * (1.5 - half * y * y)
            y = y * (1.5 - half * y * y)
            inv = y

            def put(dst, cc, val):
                plsc.store_scatter(
                    dst, [rows_r, jnp.full((16,), cc, jnp.int32)], val
                )

            put(p_v, 0, jnp.zeros((16,), jnp.float32))
            put(p_v, 1, 1000.0 * r0)
            put(p_v, 2, 1000.0 * r1)
            put(v_v, 0, -r5 * inv)
            put(v_v, 1, r3 * inv)
            put(v_v, 2, r4 * inv)
            return carry2

        lax.fori_loop(0, CHUNK // 16, group, 0)

        base = wid * BPW + ci * CHUNK
        pltpu.sync_copy(p_v, p_hbm.at[pl.ds(base, CHUNK)])
        pltpu.sync_copy(v_v, v_hbm.at[pl.ds(base, CHUNK)])
        return carry

    lax.fori_loop(0, NCHUNK, chunk_body, 0)


_sc_call = pl.kernel(
    _sc_body,
    out_type=(
        jax.ShapeDtypeStruct((N, 4), jnp.float32),
        jax.ShapeDtypeStruct((N, 4), jnp.float32),
    ),
    mesh=plsc.VectorSubcoreMesh(core_axis_name="c", subcore_axis_name="s"),
    compiler_params=pltpu.CompilerParams(
        needs_layout_passes=False, use_tc_tiling_on_sc=False
    ),
    scratch_types=[
        pltpu.VMEM((BPW // GB, GB), jnp.int32),
        pltpu.VMEM((CHUNK, D), jnp.float32),
        pltpu.VMEM((CHUNK, 4), jnp.float32),
        pltpu.VMEM((CHUNK, 4), jnp.float32),
        pltpu.SemaphoreType.DMA,
    ],
)


def kernel(all_rays, indices):
    rays8 = jnp.concatenate(
        [all_rays, jnp.zeros((all_rays.shape[0], 2), jnp.float32)], axis=1
    )
    idx2 = indices.reshape(N // GB, GB)
    p4, v4 = _sc_call(rays8, idx2)
    return (p4[:, :3], v4[:, :3])


# speedup vs baseline: 1.1962x; 1.0008x over previous
import jax
import jax.numpy as jnp
from jax import lax
from jax.experimental import pallas as pl
from jax.experimental.pallas import tpu as pltpu
from jax.experimental.pallas import tpu_sc as plsc

N = 1048576
D = 8
NC, NS = 2, 16
NW = NC * NS
BPW = N // NW
CHUNK = 2048
GB = 128
K = CHUNK // GB
NCHUNK = BPW // CHUNK


def _sc_body(rays_hbm, idx_hbm, p_hbm, v_hbm, idx_v, rows_v, p_v, v_v, gsem):
    c = lax.axis_index("c")
    s = lax.axis_index("s")
    wid = s * NC + c
    pltpu.sync_copy(idx_hbm.at[pl.ds(wid * (BPW // GB), BPW // GB)], idx_v)

    lane = lax.iota(jnp.int32, 16)

    def chunk_body(ci, carry):
        handles = []
        for j in range(K):
            handles.append(
                pltpu.async_copy(
                    rays_hbm.at[idx_v.at[ci * K + j]],
                    rows_v.at[pl.ds(j * GB, GB)],
                    gsem,
                )
            )
        for h in handles:
            h.wait()

        def group(g, carry2):
            rows_r = g * 16 + lane

            def col(cc):
                return plsc.load_gather(
                    rows_v, [rows_r, jnp.full((16,), cc, jnp.int32)]
                )

            r0 = col(0)
            r1 = col(1)
            r3 = col(3)
            r4 = col(4)
            r5 = col(5)

            ssq = r3 * r3 + r4 * r4 + r5 * r5
            seed = plsc.bitcast(
                jnp.int32(0x5F3759DF) - lax.shift_right_logical(
                    plsc.bitcast(ssq, jnp.int32), 1
                ),
                jnp.float32,
            )
            half = 0.5 * ssq
            y = seed * (1.5 - half * seed * seed)
            y = y * (1.5 - half * y * y)
            y = y * (1.5 - half * y * y)
            inv = y

            def put(dst, cc, val):
                plsc.store_scatter(
                    dst, [rows_r, jnp.full((16,), cc, jnp.int32)], val
                )

            put(p_v, 0, jnp.zeros((16,), jnp.float32))
            put(p_v, 1, 1000.0 * r0)
            put(p_v, 2, 1000.0 * r1)
            put(v_v, 0, -r5 * inv)
            put(v_v, 1, r3 * inv)
            put(v_v, 2, r4 * inv)
            return carry2

        lax.fori_loop(0, CHUNK // 16, group, 0)

        base = wid * BPW + ci * CHUNK
        pltpu.sync_copy(p_v, p_hbm.at[pl.ds(base, CHUNK)])
        pltpu.sync_copy(v_v, v_hbm.at[pl.ds(base, CHUNK)])
        return carry

    lax.fori_loop(0, NCHUNK, chunk_body, 0)


_sc_call = pl.kernel(
    _sc_body,
    out_type=(
        jax.ShapeDtypeStruct((N, 4), jnp.float32),
        jax.ShapeDtypeStruct((N, 4), jnp.float32),
    ),
    mesh=plsc.VectorSubcoreMesh(core_axis_name="c", subcore_axis_name="s"),
    compiler_params=pltpu.CompilerParams(
        needs_layout_passes=False, use_tc_tiling_on_sc=False
    ),
    scratch_types=[
        pltpu.VMEM((BPW // GB, GB), jnp.int32),
        pltpu.VMEM((CHUNK, D), jnp.float32),
        pltpu.VMEM((CHUNK, 4), jnp.float32),
        pltpu.VMEM((CHUNK, 4), jnp.float32),
        pltpu.SemaphoreType.DMA,
    ],
)


def kernel(all_rays, indices):
    rays8 = jnp.concatenate(
        [all_rays, jnp.zeros((all_rays.shape[0], 2), jnp.float32)], axis=1
    ) + 0.0
    idx2 = indices.reshape(N // GB, GB)
    p4, v4 = _sc_call(rays8, idx2)
    return (p4[:, :3] + 0.0, v4[:, :3] + 0.0)


# speedup vs baseline: 2.2430x; 1.8751x over previous
import jax
import jax.numpy as jnp
from jax import lax
from jax.experimental import pallas as pl
from jax.experimental.pallas import tpu as pltpu
from jax.experimental.pallas import tpu_sc as plsc

N = 1048576
D = 8
NC, NS = 2, 16
NW = NC * NS
BPW = N // NW
CHUNK = 2048
GB = 128
K = CHUNK // GB
NCHUNK = BPW // CHUNK


def _sc_body(rays_hbm, idx_hbm, r0_hbm, r1_hbm, vx_hbm, vy_hbm, vz_hbm,
             idx_v, rows_v, r0_v, r1_v, vx_v, vy_v, vz_v, gsem):
    c = lax.axis_index("c")
    s = lax.axis_index("s")
    wid = s * NC + c
    pltpu.sync_copy(idx_hbm.at[pl.ds(wid * (BPW // GB), BPW // GB)], idx_v)

    lane = lax.iota(jnp.int32, 16)

    def chunk_body(ci, carry):
        handles = []
        for j in range(K):
            handles.append(
                pltpu.async_copy(
                    rays_hbm.at[idx_v.at[ci * K + j]],
                    rows_v.at[pl.ds(j * GB, GB)],
                    gsem,
                )
            )
        for h in handles:
            h.wait()

        def group(g, carry2):
            rows_r = g * 16 + lane

            def col(cc):
                return plsc.load_gather(
                    rows_v, [rows_r, jnp.full((16,), cc, jnp.int32)]
                )

            r0 = col(0)
            r1 = col(1)
            r3 = col(3)
            r4 = col(4)
            r5 = col(5)

            ssq = r3 * r3 + r4 * r4 + r5 * r5
            seed = plsc.bitcast(
                jnp.int32(0x5F3759DF) - lax.shift_right_logical(
                    plsc.bitcast(ssq, jnp.int32), 1
                ),
                jnp.float32,
            )
            half = 0.5 * ssq
            y = seed * (1.5 - half * seed * seed)
            y = y * (1.5 - half * y * y)
            y = y * (1.5 - half * y * y)
            inv = y

            sl = pl.ds(g * 16, 16)
            r0_v[sl] = r0
            r1_v[sl] = r1
            vx_v[sl] = r3 * inv
            vy_v[sl] = r4 * inv
            vz_v[sl] = r5 * inv
            return carry2

        lax.fori_loop(0, CHUNK // 16, group, 0)

        base = wid * BPW + ci * CHUNK
        pltpu.sync_copy(r0_v, r0_hbm.at[pl.ds(base, CHUNK)])
        pltpu.sync_copy(r1_v, r1_hbm.at[pl.ds(base, CHUNK)])
        pltpu.sync_copy(vx_v, vx_hbm.at[pl.ds(base, CHUNK)])
        pltpu.sync_copy(vy_v, vy_hbm.at[pl.ds(base, CHUNK)])
        pltpu.sync_copy(vz_v, vz_hbm.at[pl.ds(base, CHUNK)])
        return carry

    lax.fori_loop(0, NCHUNK, chunk_body, 0)


_sc_call = pl.kernel(
    _sc_body,
    out_type=tuple(
        jax.ShapeDtypeStruct((N,), jnp.float32) for _ in range(5)
    ),
    mesh=plsc.VectorSubcoreMesh(core_axis_name="c", subcore_axis_name="s"),
    compiler_params=pltpu.CompilerParams(
        needs_layout_passes=False, use_tc_tiling_on_sc=False
    ),
    scratch_types=[
        pltpu.VMEM((BPW // GB, GB), jnp.int32),
        pltpu.VMEM((CHUNK, D), jnp.float32),
        pltpu.VMEM((CHUNK,), jnp.float32),
        pltpu.VMEM((CHUNK,), jnp.float32),
        pltpu.VMEM((CHUNK,), jnp.float32),
        pltpu.VMEM((CHUNK,), jnp.float32),
        pltpu.VMEM((CHUNK,), jnp.float32),
        pltpu.SemaphoreType.DMA,
    ],
)


def kernel(all_rays, indices):
    rays8 = jnp.concatenate(
        [all_rays, jnp.zeros((all_rays.shape[0], 2), jnp.float32)], axis=1
    )
    idx2 = indices.reshape(N // GB, GB)
    r0g, r1g, vx, vy, vz = _sc_call(rays8, idx2)
    p = jnp.stack([jnp.zeros((N,), jnp.float32), 1000.0 * r0g, 1000.0 * r1g],
                  axis=1)
    v = jnp.stack([-vz, vx, vy], axis=1)
    return (p, v)
